# baseline (speedup 1.0000x reference)
import functools

import jax
import jax.numpy as jnp
from jax import lax
from jax.experimental import pallas as pl
from jax.experimental.pallas import tpu as pltpu
from jax.experimental.pallas import tpu_sc as plsc

DIM = 50
BATCH = 16384
GAMMA = 1.0

NC = 2
NS = 16
L = 16
NW = NC * NS
BPW = BATCH // NW
CROWS = 256
NSTEP = 2 * (BPW // CROWS)
NGRP = CROWS // L
IDX_PACK = 6 * BPW


def _sc_kernel(ent_hbm, rel_hbm, idx_hbm, out_hbm,
               idx_v, h_buf, r_buf, t_buf, acc_all, sem):
  wid = lax.axis_index("s") * NC + lax.axis_index("c")
  pltpu.sync_copy(idx_hbm.at[wid], idx_v)
  iota = lax.iota(jnp.int32, L)
  tail_mask = iota >= (4 * L - DIM)

  def issue_chunk(phase, half):
    def issue_grp(g, carry):
      base = phase * (3 * BPW) + half * CROWS + g * L
      ev_h = idx_v[pl.ds(base, L)]
      ev_r = idx_v[pl.ds(base + BPW, L)]
      ev_t = idx_v[pl.ds(base + 2 * BPW, L)]
      for j in range(L):
        row = g * L + j
        pltpu.async_copy(ent_hbm.at[pl.ds(ev_h[j], 1)],
                         h_buf.at[pl.ds(row, 1)], sem)
        pltpu.async_copy(rel_hbm.at[pl.ds(ev_r[j], 1)],
                         r_buf.at[pl.ds(row, 1)], sem)
        pltpu.async_copy(ent_hbm.at[pl.ds(ev_t[j], 1)],
                         t_buf.at[pl.ds(row, 1)], sem)
      return carry
    lax.fori_loop(0, NGRP, issue_grp, jnp.int32(0))

  def drain_chunk():
    pltpu.make_async_copy(ent_hbm.at[pl.ds(0, CROWS)], h_buf, sem).wait()
    pltpu.make_async_copy(ent_hbm.at[pl.ds(0, CROWS)], r_buf, sem).wait()
    pltpu.make_async_copy(ent_hbm.at[pl.ds(0, CROWS)], t_buf, sem).wait()

  def compute_chunk(phase, half):
    def comp_grp(g, carry):
      for j in range(L):
        row = g * L + j
        acc = jnp.zeros((L,), jnp.float32)
        for k in range(3):
          sl = pl.ds(k * L, L)
          e = jnp.abs(h_buf[row, sl] + r_buf[row, sl] - t_buf[row, sl])
          acc = acc + e
        sl = pl.ds(DIM - L, L)
        e = jnp.abs(h_buf[row, sl] + r_buf[row, sl] - t_buf[row, sl])
        acc = acc + jnp.where(tail_mask, e, jnp.float32(0.0))
        off = phase * (L * BPW) + (half * CROWS + row) * L
        acc_all[pl.ds(off, L)] = acc
      return carry
    lax.fori_loop(0, NGRP, comp_grp, jnp.int32(0))

  for step in range(NSTEP):
    phase, half = step // 2, step % 2
    issue_chunk(phase, half)
    drain_chunk()
    compute_chunk(phase, half)

  def diff_q(q, carry):
    d = acc_all[pl.ds(q * L, L)] - acc_all[pl.ds(L * BPW + q * L, L)]
    acc_all[pl.ds(q * L, L)] = d
    return carry
  lax.fori_loop(0, BPW, diff_q, jnp.int32(0))
  pltpu.sync_copy(acc_all.at[pl.ds(0, L * BPW)], out_hbm.at[wid])


def _tc_finish_kernel(p_ref, o_ref):
  x = p_ref[...].reshape(NW * BPW * L // 128, 128)
  r0 = lax.broadcasted_iota(jnp.int32, (128, 128 // L), 0) // L
  r1 = lax.broadcasted_iota(jnp.int32, (128, 128 // L), 1)
  m = (r0 == r1).astype(jnp.float32)
  y = lax.dot_general(x, m, (((1,), (0,)), ((), ())),
                      preferred_element_type=jnp.float32)
  o_ref[...] = jnp.sum(
      jnp.maximum(y + jnp.float32(GAMMA), jnp.float32(0.0))
  ).reshape(1, 1)


@jax.jit
def kernel(pos_head, pos_relation, pos_tail, neg_head, neg_relation, neg_tail,
           entity_emb, relation_emb):
  packed = jnp.stack([pos_head, pos_relation, pos_tail,
                      neg_head, neg_relation, neg_tail]).astype(jnp.int32)
  packed = packed.reshape(6, NW, BPW).transpose(1, 0, 2).reshape(NW, IDX_PACK)

  mesh = plsc.VectorSubcoreMesh(core_axis_name="c", subcore_axis_name="s")
  sc = pl.kernel(
      _sc_kernel,
      out_type=jax.ShapeDtypeStruct((NW, L * BPW), jnp.float32),
      mesh=mesh,
      compiler_params=pltpu.CompilerParams(needs_layout_passes=False),
      scratch_types=[
          pltpu.VMEM((IDX_PACK,), jnp.int32),
          pltpu.VMEM((CROWS, DIM), jnp.float32),
          pltpu.VMEM((CROWS, DIM), jnp.float32),
          pltpu.VMEM((CROWS, DIM), jnp.float32),
          pltpu.VMEM((2 * L * BPW,), jnp.float32),
          pltpu.SemaphoreType.DMA,
      ],
  )
  partials = sc(entity_emb, relation_emb, packed)

  total = pl.pallas_call(
      _tc_finish_kernel,
      out_shape=jax.ShapeDtypeStruct((1, 1), jnp.float32),
  )(partials)
  return total[0, 0]
